# Initial kernel scaffold; baseline (speedup 1.0000x reference)
import jax
import jax.numpy as jnp
from jax.experimental import pallas as pl


def kernel(feature, A, weight_b, W1, b1, W2, b2):
    raise NotImplementedError("write your pallas kernel here")


# speedup vs baseline: 3.0997x; 3.0997x over previous
import jax
import jax.numpy as jnp
from jax.experimental import pallas as pl
from jax.experimental.pallas import tpu as pltpu

_BN = 512


def _pass1_kernel(w_ref, feat_ref, w1_ref, b1_ref, a_ref, t_ref, u1_ref,
                  s1_scr, u1_scr):
    i = pl.program_id(0)
    j = pl.program_id(1)
    nb = pl.num_programs(0)
    bn = a_ref.shape[1]

    @pl.when(jnp.logical_and(i == 0, j == 0))
    def _init():
        s1_scr[...] = jnp.dot(feat_ref[...], w1_ref[...],
                              preferred_element_type=jnp.float32)
        u1_scr[...] = jnp.zeros_like(u1_scr)

    m = a_ref.shape[0]
    t_blk = a_ref[0] * w_ref[0]
    for k in range(1, m):
        t_blk += a_ref[k] * w_ref[k]
    t_ref[...] = t_blk

    s1_j = s1_scr[pl.ds(j * bn, bn), :]
    s1_i = s1_scr[pl.ds(i * bn, bn), :]
    u1_scr[pl.ds(i * bn, bn), :] += jnp.dot(
        t_blk, s1_j, preferred_element_type=jnp.float32)
    u1_scr[pl.ds(j * bn, bn), :] += jax.lax.dot_general(
        t_blk, s1_i, (((0,), (0,)), ((), ())),
        preferred_element_type=jnp.float32)

    @pl.when(jnp.logical_and(i == nb - 1, j == nb - 1))
    def _fin():
        u1_ref[...] = u1_scr[...] + b1_ref[...]


def _pass2_kernel(w2_ref, b2_ref, u1_ref, t_ref, out_ref, s2_scr, x_scr):
    i = pl.program_id(0)
    j = pl.program_id(1)
    nb = pl.num_programs(0)
    bn = t_ref.shape[0]

    @pl.when(jnp.logical_and(i == 0, j == 0))
    def _init():
        s2_scr[...] = jnp.dot(u1_ref[...], w2_ref[...],
                              preferred_element_type=jnp.float32)
        x_scr[...] = jnp.zeros_like(x_scr)

    t_blk = t_ref[...]
    s2_j = s2_scr[pl.ds(j * bn, bn), :]
    s2_i = s2_scr[pl.ds(i * bn, bn), :]
    x_scr[pl.ds(i * bn, bn), :] += jnp.dot(
        t_blk, s2_j, preferred_element_type=jnp.float32)
    x_scr[pl.ds(j * bn, bn), :] += jax.lax.dot_general(
        t_blk, s2_i, (((0,), (0,)), ((), ())),
        preferred_element_type=jnp.float32)

    @pl.when(jnp.logical_and(i == nb - 1, j == nb - 1))
    def _fin():
        out_ref[...] = 0.5 * (u1_ref[...] + x_scr[...] + b2_ref[...])


@jax.jit
def kernel(feature, A, weight_b, W1, b1, W2, b2):
    n, f = feature.shape
    m = A.shape[0]
    o = W1.shape[1]
    bn = _BN
    nb = n // bn

    w = weight_b.reshape(m)
    b1r = b1.reshape(1, o)
    b2r = b2.reshape(1, o)

    t, u1 = pl.pallas_call(
        _pass1_kernel,
        grid=(nb, nb),
        in_specs=[
            pl.BlockSpec(memory_space=pltpu.SMEM),
            pl.BlockSpec((n, f), lambda i, j: (0, 0)),
            pl.BlockSpec((f, o), lambda i, j: (0, 0)),
            pl.BlockSpec((1, o), lambda i, j: (0, 0)),
            pl.BlockSpec((m, bn, bn), lambda i, j: (0, i, j)),
        ],
        out_specs=[
            pl.BlockSpec((bn, bn), lambda i, j: (i, j)),
            pl.BlockSpec((n, o), lambda i, j: (0, 0)),
        ],
        out_shape=[
            jax.ShapeDtypeStruct((n, n), jnp.float32),
            jax.ShapeDtypeStruct((n, o), jnp.float32),
        ],
        scratch_shapes=[
            pltpu.VMEM((n, o), jnp.float32),
            pltpu.VMEM((n, o), jnp.float32),
        ],
    )(w, feature, W1, b1r, A)

    out = pl.pallas_call(
        _pass2_kernel,
        grid=(nb, nb),
        in_specs=[
            pl.BlockSpec((o, o), lambda i, j: (0, 0)),
            pl.BlockSpec((1, o), lambda i, j: (0, 0)),
            pl.BlockSpec((n, o), lambda i, j: (0, 0)),
            pl.BlockSpec((bn, bn), lambda i, j: (i, j)),
        ],
        out_specs=pl.BlockSpec((n, o), lambda i, j: (0, 0)),
        out_shape=jax.ShapeDtypeStruct((n, o), jnp.float32),
        scratch_shapes=[
            pltpu.VMEM((n, o), jnp.float32),
            pltpu.VMEM((n, o), jnp.float32),
        ],
    )(W2, b2r, u1, t)

    return out


# speedup vs baseline: 3.3788x; 1.0900x over previous
import jax
import jax.numpy as jnp
from jax.experimental import pallas as pl
from jax.experimental.pallas import tpu as pltpu

_BN = 512


def _pass1_kernel(w_ref, feat_ref, w1_ref, b1_ref, a_ref, t_ref, u1_ref,
                  s1_scr, u1_scr):
    i = pl.program_id(0)
    j = pl.program_id(1)
    nb = pl.num_programs(0)
    bn = a_ref.shape[1]

    @pl.when(jnp.logical_and(i == 0, j == 0))
    def _init():
        s1_scr[...] = jnp.dot(feat_ref[...], w1_ref[...],
                              preferred_element_type=jnp.float32)
        u1_scr[...] = jnp.zeros_like(u1_scr)

    m = a_ref.shape[0]
    t_blk = a_ref[0] * w_ref[0]
    for k in range(1, m):
        t_blk += a_ref[k] * w_ref[k]
    t_ref[...] = t_blk.astype(jnp.bfloat16)

    s1_j = s1_scr[pl.ds(j * bn, bn), :]
    s1_i = s1_scr[pl.ds(i * bn, bn), :]
    u1_scr[pl.ds(i * bn, bn), :] += jnp.dot(
        t_blk, s1_j, preferred_element_type=jnp.float32)
    u1_scr[pl.ds(j * bn, bn), :] += jax.lax.dot_general(
        t_blk, s1_i, (((0,), (0,)), ((), ())),
        preferred_element_type=jnp.float32)

    @pl.when(jnp.logical_and(i == nb - 1, j == nb - 1))
    def _fin():
        u1_ref[...] = u1_scr[...] + b1_ref[...]


def _pass2_kernel(w2_ref, b2_ref, u1_ref, t_ref, out_ref, s2_scr, x_scr):
    i = pl.program_id(0)
    j = pl.program_id(1)
    nb = pl.num_programs(0)
    bn = t_ref.shape[0]

    @pl.when(jnp.logical_and(i == 0, j == 0))
    def _init():
        s2_scr[...] = jnp.dot(u1_ref[...], w2_ref[...],
                              preferred_element_type=jnp.float32
                              ).astype(jnp.bfloat16)
        x_scr[...] = jnp.zeros_like(x_scr)

    t_blk = t_ref[...]
    s2_j = s2_scr[pl.ds(j * bn, bn), :]
    s2_i = s2_scr[pl.ds(i * bn, bn), :]
    x_scr[pl.ds(i * bn, bn), :] += jnp.dot(
        t_blk, s2_j, preferred_element_type=jnp.float32)
    x_scr[pl.ds(j * bn, bn), :] += jax.lax.dot_general(
        t_blk, s2_i, (((0,), (0,)), ((), ())),
        preferred_element_type=jnp.float32)

    @pl.when(jnp.logical_and(i == nb - 1, j == nb - 1))
    def _fin():
        out_ref[...] = 0.5 * (u1_ref[...] + x_scr[...] + b2_ref[...])


@jax.jit
def kernel(feature, A, weight_b, W1, b1, W2, b2):
    n, f = feature.shape
    m = A.shape[0]
    o = W1.shape[1]
    bn = _BN
    nb = n // bn

    w = weight_b.reshape(m)
    b1r = b1.reshape(1, o)
    b2r = b2.reshape(1, o)

    t, u1 = pl.pallas_call(
        _pass1_kernel,
        grid=(nb, nb),
        in_specs=[
            pl.BlockSpec(memory_space=pltpu.SMEM),
            pl.BlockSpec((n, f), lambda i, j: (0, 0)),
            pl.BlockSpec((f, o), lambda i, j: (0, 0)),
            pl.BlockSpec((1, o), lambda i, j: (0, 0)),
            pl.BlockSpec((m, bn, bn), lambda i, j: (0, i, j)),
        ],
        out_specs=[
            pl.BlockSpec((bn, bn), lambda i, j: (i, j)),
            pl.BlockSpec((n, o), lambda i, j: (0, 0)),
        ],
        out_shape=[
            jax.ShapeDtypeStruct((n, n), jnp.bfloat16),
            jax.ShapeDtypeStruct((n, o), jnp.float32),
        ],
        scratch_shapes=[
            pltpu.VMEM((n, o), jnp.float32),
            pltpu.VMEM((n, o), jnp.float32),
        ],
    )(w, feature, W1, b1r, A)

    out = pl.pallas_call(
        _pass2_kernel,
        grid=(nb, nb),
        in_specs=[
            pl.BlockSpec((o, o), lambda i, j: (0, 0)),
            pl.BlockSpec((1, o), lambda i, j: (0, 0)),
            pl.BlockSpec((n, o), lambda i, j: (0, 0)),
            pl.BlockSpec((bn, bn), lambda i, j: (i, j)),
        ],
        out_specs=pl.BlockSpec((n, o), lambda i, j: (0, 0)),
        out_shape=jax.ShapeDtypeStruct((n, o), jnp.float32),
        scratch_shapes=[
            pltpu.VMEM((n, o), jnp.bfloat16),
            pltpu.VMEM((n, o), jnp.float32),
        ],
    )(W2, b2r, u1, t)

    return out


# speedup vs baseline: 4.1558x; 1.2300x over previous
import jax
import jax.numpy as jnp
from jax.experimental import pallas as pl
from jax.experimental.pallas import tpu as pltpu

_BN = 512


def _pass1_kernel(w_ref, feat_ref, w1_ref, b1_ref, a_ref, t_ref, u1_ref,
                  s1_scr, u1_scr):
    i = pl.program_id(0)
    j = pl.program_id(1)
    nb = pl.num_programs(0)
    bn = a_ref.shape[1]

    @pl.when(jnp.logical_and(i == 0, j == 0))
    def _init():
        s1_scr[...] = jnp.dot(feat_ref[...], w1_ref[...],
                              preferred_element_type=jnp.float32)
        u1_scr[...] = jnp.zeros_like(u1_scr)

    m = a_ref.shape[0]
    t_blk = a_ref[0] * w_ref[0]
    for k in range(1, m):
        t_blk += a_ref[k] * w_ref[k]
    t_ref[...] = t_blk.astype(jnp.bfloat16)

    s1_j = s1_scr[pl.ds(j * bn, bn), :]
    s1_i = s1_scr[pl.ds(i * bn, bn), :]
    u1_scr[pl.ds(i * bn, bn), :] += jnp.dot(
        t_blk, s1_j, preferred_element_type=jnp.float32)
    u1_scr[pl.ds(j * bn, bn), :] += jax.lax.dot_general(
        t_blk, s1_i, (((0,), (0,)), ((), ())),
        preferred_element_type=jnp.float32)

    @pl.when(jnp.logical_and(i == nb - 1, j == nb - 1))
    def _fin():
        u1_ref[...] = u1_scr[...] + b1_ref[...]


def _pass2_kernel(w2_ref, b2_ref, u1_ref, t_ref, out_ref, s2_scr, x_scr):
    i = pl.program_id(0)
    nb = pl.num_programs(0)
    bn = t_ref.shape[0]

    @pl.when(i == 0)
    def _init():
        s2_scr[...] = jnp.dot(u1_ref[...], w2_ref[...],
                              preferred_element_type=jnp.float32
                              ).astype(jnp.bfloat16)
        x_scr[...] = jnp.zeros_like(x_scr)

    t_strip = t_ref[...]
    s2_i = s2_scr[pl.ds(i * bn, bn), :]
    x_scr[pl.ds(i * bn, bn), :] += jnp.dot(
        t_strip, s2_scr[...], preferred_element_type=jnp.float32)
    x_scr[...] += jax.lax.dot_general(
        t_strip, s2_i, (((0,), (0,)), ((), ())),
        preferred_element_type=jnp.float32)

    @pl.when(i == nb - 1)
    def _fin():
        out_ref[...] = 0.5 * (u1_ref[...] + x_scr[...] + b2_ref[...])


@jax.jit
def kernel(feature, A, weight_b, W1, b1, W2, b2):
    n, f = feature.shape
    m = A.shape[0]
    o = W1.shape[1]
    bn = _BN
    nb = n // bn

    w = weight_b.reshape(m)
    b1r = b1.reshape(1, o)
    b2r = b2.reshape(1, o)

    t, u1 = pl.pallas_call(
        _pass1_kernel,
        grid=(nb, nb),
        in_specs=[
            pl.BlockSpec(memory_space=pltpu.SMEM),
            pl.BlockSpec((n, f), lambda i, j: (0, 0)),
            pl.BlockSpec((f, o), lambda i, j: (0, 0)),
            pl.BlockSpec((1, o), lambda i, j: (0, 0)),
            pl.BlockSpec((m, bn, bn), lambda i, j: (0, i, j)),
        ],
        out_specs=[
            pl.BlockSpec((bn, bn), lambda i, j: (i, j)),
            pl.BlockSpec((n, o), lambda i, j: (0, 0)),
        ],
        out_shape=[
            jax.ShapeDtypeStruct((n, n), jnp.bfloat16),
            jax.ShapeDtypeStruct((n, o), jnp.float32),
        ],
        scratch_shapes=[
            pltpu.VMEM((n, o), jnp.float32),
            pltpu.VMEM((n, o), jnp.float32),
        ],
    )(w, feature, W1, b1r, A)

    out = pl.pallas_call(
        _pass2_kernel,
        grid=(nb,),
        in_specs=[
            pl.BlockSpec((o, o), lambda i: (0, 0)),
            pl.BlockSpec((1, o), lambda i: (0, 0)),
            pl.BlockSpec((n, o), lambda i: (0, 0)),
            pl.BlockSpec((bn, n), lambda i: (i, 0)),
        ],
        out_specs=pl.BlockSpec((n, o), lambda i: (0, 0)),
        out_shape=jax.ShapeDtypeStruct((n, o), jnp.float32),
        scratch_shapes=[
            pltpu.VMEM((n, o), jnp.bfloat16),
            pltpu.VMEM((n, o), jnp.float32),
        ],
    )(W2, b2r, u1, t)

    return out


# speedup vs baseline: 4.5268x; 1.0893x over previous
import jax
import jax.numpy as jnp
from jax.experimental import pallas as pl
from jax.experimental.pallas import tpu as pltpu

_BN = 512


def _mhgcn_kernel(w_ref, feat_ref, w1_ref, b1_ref, w2_ref, b2_ref, a_ref,
                  out_ref, t_scr, s1_scr, s2_scr, u1_scr, x_scr):
    s = pl.program_id(0)
    ns = pl.num_programs(0)
    m, bn, _ = a_ref.shape
    n = t_scr.shape[0]
    nb = n // bn
    nsq = nb * nb

    @pl.when(s == 0)
    def _init():
        s1_scr[...] = jnp.dot(feat_ref[...], w1_ref[...],
                              preferred_element_type=jnp.float32)
        u1_scr[...] = jnp.zeros_like(u1_scr)
        x_scr[...] = jnp.zeros_like(x_scr)

    @pl.when(s < nsq)
    def _phase0():
        i = s // nb
        j = s - i * nb
        t_blk = a_ref[0] * w_ref[0]
        for k in range(1, m):
            t_blk += a_ref[k] * w_ref[k]
        t_scr[pl.ds(i * bn, bn), pl.ds(j * bn, bn)] = t_blk.astype(
            jnp.bfloat16)
        s1_j = s1_scr[pl.ds(j * bn, bn), :]
        s1_i = s1_scr[pl.ds(i * bn, bn), :]
        u1_scr[pl.ds(i * bn, bn), :] += jnp.dot(
            t_blk, s1_j, preferred_element_type=jnp.float32)
        u1_scr[pl.ds(j * bn, bn), :] += jax.lax.dot_general(
            t_blk, s1_i, (((0,), (0,)), ((), ())),
            preferred_element_type=jnp.float32)

    @pl.when(s == nsq - 1)
    def _mid():
        u1_scr[...] += b1_ref[...]
        s2_scr[...] = jnp.dot(u1_scr[...], w2_ref[...],
                              preferred_element_type=jnp.float32
                              ).astype(jnp.bfloat16)

    @pl.when(s >= nsq)
    def _phase1():
        k = s - nsq
        strip = t_scr[pl.ds(k * bn, bn), :]
        s2_k = s2_scr[pl.ds(k * bn, bn), :]
        x_scr[pl.ds(k * bn, bn), :] += jnp.dot(
            strip, s2_scr[...], preferred_element_type=jnp.float32)
        x_scr[...] += jax.lax.dot_general(
            strip, s2_k, (((0,), (0,)), ((), ())),
            preferred_element_type=jnp.float32)

    @pl.when(s == ns - 1)
    def _fin():
        out_ref[...] = 0.5 * (u1_scr[...] + x_scr[...] + b2_ref[...])


@jax.jit
def kernel(feature, A, weight_b, W1, b1, W2, b2):
    n, f = feature.shape
    m = A.shape[0]
    o = W1.shape[1]
    bn = _BN
    nb = n // bn
    nsq = nb * nb

    w = weight_b.reshape(m)
    b1r = b1.reshape(1, o)
    b2r = b2.reshape(1, o)

    def a_map(s):
        sc = jnp.minimum(s, nsq - 1)
        return (0, sc // nb, sc % nb)

    out = pl.pallas_call(
        _mhgcn_kernel,
        grid=(nsq + nb,),
        in_specs=[
            pl.BlockSpec(memory_space=pltpu.SMEM),
            pl.BlockSpec((n, f), lambda s: (0, 0)),
            pl.BlockSpec((f, o), lambda s: (0, 0)),
            pl.BlockSpec((1, o), lambda s: (0, 0)),
            pl.BlockSpec((o, o), lambda s: (0, 0)),
            pl.BlockSpec((1, o), lambda s: (0, 0)),
            pl.BlockSpec((m, bn, bn), a_map),
        ],
        out_specs=pl.BlockSpec((n, o), lambda s: (0, 0)),
        out_shape=jax.ShapeDtypeStruct((n, o), jnp.float32),
        scratch_shapes=[
            pltpu.VMEM((n, n), jnp.bfloat16),
            pltpu.VMEM((n, o), jnp.float32),
            pltpu.VMEM((n, o), jnp.bfloat16),
            pltpu.VMEM((n, o), jnp.float32),
            pltpu.VMEM((n, o), jnp.float32),
        ],
    )(w, feature, W1, b1r, W2, b2r, A)

    return out


# speedup vs baseline: 4.5535x; 1.0059x over previous
import jax
import jax.numpy as jnp
from jax.experimental import pallas as pl
from jax.experimental.pallas import tpu as pltpu

_BN = 512


def _mhgcn_kernel(w_ref, feat_ref, w1_ref, b1_ref, w2_ref, b2_ref, a_ref,
                  out_ref, t_scr, s1_scr, s2_scr, u1_scr, x_scr):
    s = pl.program_id(0)
    ns = pl.num_programs(0)
    m, bn, _ = a_ref.shape
    n = t_scr.shape[0]
    nb = n // bn
    nsq = nb * nb

    @pl.when(s == 0)
    def _init():
        s1_scr[...] = jnp.dot(feat_ref[...], w1_ref[...],
                              preferred_element_type=jnp.float32
                              ).astype(jnp.bfloat16)
        u1_scr[...] = jnp.zeros_like(u1_scr)
        x_scr[...] = jnp.zeros_like(x_scr)

    @pl.when(s < nsq)
    def _phase0():
        i = s // nb
        j = s - i * nb
        t_blk = a_ref[0] * w_ref[0]
        for k in range(1, m):
            t_blk += a_ref[k] * w_ref[k]
        tb = t_blk.astype(jnp.bfloat16)
        t_scr[pl.ds(i * bn, bn), pl.ds(j * bn, bn)] = tb
        s1_j = s1_scr[pl.ds(j * bn, bn), :]
        s1_i = s1_scr[pl.ds(i * bn, bn), :]
        u1_scr[pl.ds(i * bn, bn), :] += jnp.dot(
            tb, s1_j, preferred_element_type=jnp.float32)
        u1_scr[pl.ds(j * bn, bn), :] += jax.lax.dot_general(
            tb, s1_i, (((0,), (0,)), ((), ())),
            preferred_element_type=jnp.float32)

    @pl.when(s == nsq - 1)
    def _mid():
        u1_scr[...] += b1_ref[...]
        s2_scr[...] = jnp.dot(u1_scr[...], w2_ref[...],
                              preferred_element_type=jnp.float32
                              ).astype(jnp.bfloat16)

    @pl.when(s >= nsq)
    def _phase1():
        k = s - nsq
        strip = t_scr[pl.ds(k * bn, bn), :]
        s2_k = s2_scr[pl.ds(k * bn, bn), :]
        x_scr[pl.ds(k * bn, bn), :] += jnp.dot(
            strip, s2_scr[...], preferred_element_type=jnp.float32)
        x_scr[...] += jax.lax.dot_general(
            strip, s2_k, (((0,), (0,)), ((), ())),
            preferred_element_type=jnp.float32)

    @pl.when(s == ns - 1)
    def _fin():
        out_ref[...] = 0.5 * (u1_scr[...] + x_scr[...] + b2_ref[...])


@jax.jit
def kernel(feature, A, weight_b, W1, b1, W2, b2):
    n, f = feature.shape
    m = A.shape[0]
    o = W1.shape[1]
    bn = _BN
    nb = n // bn
    nsq = nb * nb

    w = weight_b.reshape(m)
    b1r = b1.reshape(1, o)
    b2r = b2.reshape(1, o)

    def a_map(s):
        sc = jnp.minimum(s, nsq - 1)
        return (0, sc // nb, sc % nb)

    out = pl.pallas_call(
        _mhgcn_kernel,
        grid=(nsq + nb,),
        in_specs=[
            pl.BlockSpec(memory_space=pltpu.SMEM),
            pl.BlockSpec((n, f), lambda s: (0, 0)),
            pl.BlockSpec((f, o), lambda s: (0, 0)),
            pl.BlockSpec((1, o), lambda s: (0, 0)),
            pl.BlockSpec((o, o), lambda s: (0, 0)),
            pl.BlockSpec((1, o), lambda s: (0, 0)),
            pl.BlockSpec((m, bn, bn), a_map),
        ],
        out_specs=pl.BlockSpec((n, o), lambda s: (0, 0)),
        out_shape=jax.ShapeDtypeStruct((n, o), jnp.float32),
        scratch_shapes=[
            pltpu.VMEM((n, n), jnp.bfloat16),
            pltpu.VMEM((n, o), jnp.bfloat16),
            pltpu.VMEM((n, o), jnp.bfloat16),
            pltpu.VMEM((n, o), jnp.float32),
            pltpu.VMEM((n, o), jnp.float32),
        ],
    )(w, feature, W1, b1r, W2, b2r, A)

    return out


# speedup vs baseline: 5.6271x; 1.2358x over previous
import jax
import jax.numpy as jnp
from jax.experimental import pallas as pl
from jax.experimental.pallas import tpu as pltpu

_BN = 512
_BNR = 128


def _mhgcn_kernel(w_ref, feat_ref, w1_ref, b1_ref, w2_ref, b2_ref, a_ref,
                  out_ref, t_scr, s1_scr, s2_scr, u1_scr, x_scr):
    s = pl.program_id(0)
    ns = pl.num_programs(0)
    m, bnr, n = a_ref.shape
    nr = n // bnr
    bn = _BN

    @pl.when(s == 0)
    def _init():
        s1_scr[...] = jnp.dot(feat_ref[...], w1_ref[...],
                              preferred_element_type=jnp.float32
                              ).astype(jnp.bfloat16)
        u1_scr[...] = jnp.zeros_like(u1_scr)
        x_scr[...] = jnp.zeros_like(x_scr)

    @pl.when(s < nr)
    def _phase0():
        t_strip = a_ref[0] * w_ref[0]
        for k in range(1, m):
            t_strip += a_ref[k] * w_ref[k]
        tb = t_strip.astype(jnp.bfloat16)
        t_scr[pl.ds(s * bnr, bnr), :] = tb
        s1_s = s1_scr[pl.ds(s * bnr, bnr), :]
        u1_scr[pl.ds(s * bnr, bnr), :] += jnp.dot(
            tb, s1_scr[...], preferred_element_type=jnp.float32)
        u1_scr[...] += jax.lax.dot_general(
            tb, s1_s, (((0,), (0,)), ((), ())),
            preferred_element_type=jnp.float32)

    @pl.when(s == nr - 1)
    def _mid():
        u1_scr[...] += b1_ref[...]
        s2_scr[...] = jnp.dot(u1_scr[...], w2_ref[...],
                              preferred_element_type=jnp.float32
                              ).astype(jnp.bfloat16)

    @pl.when(s >= nr)
    def _phase1():
        k = s - nr
        strip = t_scr[pl.ds(k * bn, bn), :]
        s2_k = s2_scr[pl.ds(k * bn, bn), :]
        x_scr[pl.ds(k * bn, bn), :] += jnp.dot(
            strip, s2_scr[...], preferred_element_type=jnp.float32)
        x_scr[...] += jax.lax.dot_general(
            strip, s2_k, (((0,), (0,)), ((), ())),
            preferred_element_type=jnp.float32)

    @pl.when(s == ns - 1)
    def _fin():
        out_ref[...] = 0.5 * (u1_scr[...] + x_scr[...] + b2_ref[...])


@jax.jit
def kernel(feature, A, weight_b, W1, b1, W2, b2):
    n, f = feature.shape
    m = A.shape[0]
    o = W1.shape[1]
    bn = _BN
    nb = n // bn
    bnr = _BNR
    nr = n // bnr

    w = weight_b.reshape(m)
    b1r = b1.reshape(1, o)
    b2r = b2.reshape(1, o)

    def a_map(s):
        return (0, jnp.minimum(s, nr - 1), 0)

    out = pl.pallas_call(
        _mhgcn_kernel,
        grid=(nr + nb,),
        in_specs=[
            pl.BlockSpec(memory_space=pltpu.SMEM),
            pl.BlockSpec((n, f), lambda s: (0, 0)),
            pl.BlockSpec((f, o), lambda s: (0, 0)),
            pl.BlockSpec((1, o), lambda s: (0, 0)),
            pl.BlockSpec((o, o), lambda s: (0, 0)),
            pl.BlockSpec((1, o), lambda s: (0, 0)),
            pl.BlockSpec((m, bnr, n), a_map),
        ],
        out_specs=pl.BlockSpec((n, o), lambda s: (0, 0)),
        out_shape=jax.ShapeDtypeStruct((n, o), jnp.float32),
        scratch_shapes=[
            pltpu.VMEM((n, n), jnp.bfloat16),
            pltpu.VMEM((n, o), jnp.bfloat16),
            pltpu.VMEM((n, o), jnp.bfloat16),
            pltpu.VMEM((n, o), jnp.float32),
            pltpu.VMEM((n, o), jnp.float32),
        ],
    )(w, feature, W1, b1r, W2, b2r, A)

    return out
